# baseline (speedup 1.0000x reference)
import functools

import jax
import jax.numpy as jnp
from jax import lax
from jax.experimental import pallas as pl
from jax.experimental.pallas import tpu as pltpu
from jax.experimental.pallas import tpu_sc as plsc

_NUM_CORES = 2
_NUM_SUBCORES = 16
_NUM_WORKERS = _NUM_CORES * _NUM_SUBCORES
_LANES = 16


def _emb_lookup(labels_flat, wte, wpe, seq_len):
    n = labels_flat.shape[0]
    _, d = wte.shape
    per_w = n // _NUM_WORKERS
    chunk = 16
    n_chunks = per_w // chunk
    workers_per_seq = seq_len // per_w

    mesh = plsc.VectorSubcoreMesh(core_axis_name="c", subcore_axis_name="s")

    @functools.partial(
        pl.kernel,
        mesh=mesh,
        out_type=jax.ShapeDtypeStruct((n, d), jnp.float32),
        scratch_types=[
            pltpu.VMEM((per_w,), jnp.int32),
            pltpu.VMEM((chunk, d), jnp.float32),
            pltpu.VMEM((chunk, d), jnp.float32),
            pltpu.SemaphoreType.DMA,
        ],
    )
    def emb_kernel(wte_hbm, idx_hbm, wpe_hbm, out_hbm, idx_v, rows_v, wpe_v, sem):
        wid = lax.axis_index("s") * _NUM_CORES + lax.axis_index("c")
        base = wid * per_w
        seq_base = (wid % workers_per_seq) * per_w

        pltpu.sync_copy(idx_hbm.at[pl.ds(base, per_w)], idx_v)

        @pl.loop(0, n_chunks)
        def _chunk_loop(c):
            pltpu.async_copy(
                wte_hbm.at[idx_v.at[pl.ds(c * chunk, chunk)]], rows_v, sem
            ).wait()
            pltpu.sync_copy(wpe_hbm.at[pl.ds(seq_base + c * chunk, chunk)], wpe_v)

            @pl.loop(0, chunk)
            def _row_loop(r):
                @pl.loop(0, d, step=_LANES)
                def _lane_loop(j):
                    rows_v[r, pl.ds(j, _LANES)] = (
                        rows_v[r, pl.ds(j, _LANES)] + wpe_v[r, pl.ds(j, _LANES)]
                    )

            pltpu.sync_copy(rows_v, out_hbm.at[pl.ds(base + c * chunk, chunk)])

    return emb_kernel(wte, labels_flat, wpe)


def kernel(hidden, labels, wte, wpe):
    b, s = labels.shape
    d = wte.shape[1]
    labels_flat = labels.reshape(b * s)
    emb = _emb_lookup(labels_flat, wte, wpe, s).reshape(b, s, d)
    attention_mask = labels != 0
    return (hidden, emb, labels, attention_mask)


# speedup vs baseline: 1.8916x; 1.8916x over previous
import functools

import jax
import jax.numpy as jnp
from jax import lax
from jax.experimental import pallas as pl
from jax.experimental.pallas import tpu as pltpu
from jax.experimental.pallas import tpu_sc as plsc

_NUM_CORES = 2
_NUM_SUBCORES = 16
_NUM_WORKERS = _NUM_CORES * _NUM_SUBCORES
_LANES = 16
_NBUF = 3
_CHUNK = 16


def _emb_lookup(labels_flat, wte, wpe, seq_len):
    n = labels_flat.shape[0]
    _, d = wte.shape
    per_w = n // _NUM_WORKERS
    n_chunks = per_w // _CHUNK
    workers_per_seq = seq_len // per_w

    mesh = plsc.VectorSubcoreMesh(core_axis_name="c", subcore_axis_name="s")

    @functools.partial(
        pl.kernel,
        mesh=mesh,
        out_type=jax.ShapeDtypeStruct((n, d), jnp.float32),
        scratch_types=(
            [pltpu.VMEM((per_w,), jnp.int32)]
            + [pltpu.VMEM((_CHUNK, d), jnp.float32) for _ in range(2 * _NBUF)]
            + [pltpu.SemaphoreType.DMA for _ in range(2 * _NBUF)]
        ),
    )
    def emb_kernel(wte_hbm, idx_hbm, wpe_hbm, out_hbm, idx_v, *bufs_and_sems):
        rows = bufs_and_sems[0:_NBUF]
        wpes = bufs_and_sems[_NBUF:2 * _NBUF]
        isems = bufs_and_sems[2 * _NBUF:3 * _NBUF]
        osems = bufs_and_sems[3 * _NBUF:4 * _NBUF]

        wid = lax.axis_index("s") * _NUM_CORES + lax.axis_index("c")
        base = wid * per_w
        seq_base = (wid % workers_per_seq) * per_w

        pltpu.sync_copy(idx_hbm.at[pl.ds(base, per_w)], idx_v)

        def in_copies(c, b):
            gat = pltpu.make_async_copy(
                wte_hbm.at[idx_v.at[pl.ds(c * _CHUNK, _CHUNK)]], rows[b], isems[b]
            )
            pos = pltpu.make_async_copy(
                wpe_hbm.at[pl.ds(seq_base + c * _CHUNK, _CHUNK)], wpes[b], isems[b]
            )
            return gat, pos

        def out_copy(c, b):
            return pltpu.make_async_copy(
                rows[b], out_hbm.at[pl.ds(base + c * _CHUNK, _CHUNK)], osems[b]
            )

        for step in range(n_chunks + 1):
            if step < n_chunks:
                b = step % _NBUF
                if step >= _NBUF:
                    out_copy(step - _NBUF, b).wait()
                gat, pos = in_copies(step, b)
                gat.start()
                pos.start()
            if step >= 1:
                c = step - 1
                b = c % _NBUF
                gat, pos = in_copies(c, b)
                gat.wait()
                pos.wait()

                @pl.loop(0, _CHUNK)
                def _row_loop(r, _b=b):
                    rv, wv = rows[_b], wpes[_b]
                    for j in range(0, d, _LANES):
                        rv[r, pl.ds(j, _LANES)] = (
                            rv[r, pl.ds(j, _LANES)] + wv[r, pl.ds(j, _LANES)]
                        )

                out_copy(c, b).start()

        for c in range(n_chunks - _NBUF, n_chunks):
            out_copy(c, c % _NBUF).wait()

    return emb_kernel(wte, labels_flat, wpe)


def kernel(hidden, labels, wte, wpe):
    b, s = labels.shape
    d = wte.shape[1]
    labels_flat = labels.reshape(b * s)
    emb = _emb_lookup(labels_flat, wte, wpe, s).reshape(b, s, d)
    attention_mask = labels != 0
    return (hidden, emb, labels, attention_mask)


# speedup vs baseline: 1.9038x; 1.0064x over previous
import functools

import jax
import jax.numpy as jnp
from jax import lax
from jax.experimental import pallas as pl
from jax.experimental.pallas import tpu as pltpu
from jax.experimental.pallas import tpu_sc as plsc

_NUM_CORES = 2
_NUM_SUBCORES = 16
_NUM_WORKERS = _NUM_CORES * _NUM_SUBCORES
_LANES = 16
_NBUF = 3
_CHUNK = 16


def _emb_lookup(labels_flat, wte, wpe, batch, seq_len):
    n = labels_flat.shape[0]
    _, d = wte.shape
    per_w = n // _NUM_WORKERS
    n_chunks = per_w // _CHUNK
    workers_per_seq = seq_len // per_w

    mesh = plsc.VectorSubcoreMesh(core_axis_name="c", subcore_axis_name="s")

    @functools.partial(
        pl.kernel,
        mesh=mesh,
        out_type=jax.ShapeDtypeStruct((batch, seq_len, d), jnp.float32),
        scratch_types=(
            [pltpu.VMEM((per_w,), jnp.int32)]
            + [pltpu.VMEM((_CHUNK, d), jnp.float32) for _ in range(2 * _NBUF)]
            + [pltpu.SemaphoreType.DMA for _ in range(2 * _NBUF)]
        ),
    )
    def emb_kernel(wte_hbm, idx_hbm, wpe_hbm, out_hbm, idx_v, *bufs_and_sems):
        rows = bufs_and_sems[0:_NBUF]
        wpes = bufs_and_sems[_NBUF:2 * _NBUF]
        isems = bufs_and_sems[2 * _NBUF:3 * _NBUF]
        osems = bufs_and_sems[3 * _NBUF:4 * _NBUF]

        wid = lax.axis_index("s") * _NUM_CORES + lax.axis_index("c")
        base = wid * per_w
        seq_base = (wid % workers_per_seq) * per_w
        batch_idx = wid // workers_per_seq

        pltpu.sync_copy(idx_hbm.at[pl.ds(base, per_w)], idx_v)

        def in_copies(c, b):
            gat = pltpu.make_async_copy(
                wte_hbm.at[idx_v.at[pl.ds(c * _CHUNK, _CHUNK)]], rows[b], isems[b]
            )
            pos = pltpu.make_async_copy(
                wpe_hbm.at[pl.ds(seq_base + c * _CHUNK, _CHUNK)], wpes[b], isems[b]
            )
            return gat, pos

        def out_copy(c, b):
            return pltpu.make_async_copy(
                rows[b],
                out_hbm.at[batch_idx, pl.ds(seq_base + c * _CHUNK, _CHUNK)],
                osems[b],
            )

        for step in range(n_chunks + 1):
            if step < n_chunks:
                b = step % _NBUF
                if step >= _NBUF:
                    out_copy(step - _NBUF, b).wait()
                gat, pos = in_copies(step, b)
                gat.start()
                pos.start()
            if step >= 1:
                c = step - 1
                b = c % _NBUF
                gat, pos = in_copies(c, b)
                gat.wait()
                pos.wait()

                @pl.loop(0, _CHUNK)
                def _row_loop(r, _b=b):
                    rv, wv = rows[_b], wpes[_b]
                    for j in range(0, d, _LANES):
                        rv[r, pl.ds(j, _LANES)] = (
                            rv[r, pl.ds(j, _LANES)] + wv[r, pl.ds(j, _LANES)]
                        )

                out_copy(c, b).start()

        for c in range(n_chunks - _NBUF, n_chunks):
            out_copy(c, c % _NBUF).wait()

    return emb_kernel(wte, labels_flat, wpe)


def kernel(hidden, labels, wte, wpe):
    b, s = labels.shape
    labels_flat = labels.reshape(b * s)
    emb = _emb_lookup(labels_flat, wte, wpe, b, s)
    attention_mask = labels != 0
    return (hidden, emb, labels, attention_mask)


# speedup vs baseline: 2.2253x; 1.1689x over previous
import functools

import jax
import jax.numpy as jnp
from jax import lax
from jax.experimental import pallas as pl
from jax.experimental.pallas import tpu as pltpu
from jax.experimental.pallas import tpu_sc as plsc

_NUM_CORES = 2
_NUM_SUBCORES = 16
_NUM_WORKERS = _NUM_CORES * _NUM_SUBCORES
_LANES = 16
_NBUF = 3
_CHUNK = 16


def _emb_lookup(labels_flat, wte, wpe, batch, seq_len):
    n = labels_flat.shape[0]
    _, d = wte.shape
    pos_per_w = seq_len // _NUM_WORKERS
    per_w = n // _NUM_WORKERS
    n_chunks = per_w // _CHUNK
    chunks_per_seq = pos_per_w // _CHUNK

    mesh = plsc.VectorSubcoreMesh(core_axis_name="c", subcore_axis_name="s")

    @functools.partial(
        pl.kernel,
        mesh=mesh,
        out_type=jax.ShapeDtypeStruct((batch, seq_len, d), jnp.float32),
        scratch_types=(
            [pltpu.VMEM((per_w,), jnp.int32),
             pltpu.VMEM((pos_per_w, d), jnp.float32)]
            + [pltpu.VMEM((_CHUNK, d), jnp.float32) for _ in range(_NBUF)]
            + [pltpu.SemaphoreType.DMA for _ in range(2 * _NBUF)]
        ),
    )
    def emb_kernel(wte_hbm, idx_hbm, wpe_hbm, out_hbm, idx_v, wpe_v, *bufs_and_sems):
        rows = bufs_and_sems[0:_NBUF]
        isems = bufs_and_sems[_NBUF:2 * _NBUF]
        osems = bufs_and_sems[2 * _NBUF:3 * _NBUF]

        wid = lax.axis_index("s") * _NUM_CORES + lax.axis_index("c")
        pos_base = wid * pos_per_w

        pltpu.sync_copy(wpe_hbm.at[pl.ds(pos_base, pos_per_w)], wpe_v)
        for b_idx in range(batch):
            pltpu.sync_copy(
                idx_hbm.at[pl.ds(b_idx * seq_len + pos_base, pos_per_w)],
                idx_v.at[pl.ds(b_idx * pos_per_w, pos_per_w)],
            )

        def gather(c, b):
            return pltpu.make_async_copy(
                wte_hbm.at[idx_v.at[pl.ds(c * _CHUNK, _CHUNK)]], rows[b], isems[b]
            )

        def out_copy(c, b):
            b_idx, cc = divmod(c, chunks_per_seq)
            return pltpu.make_async_copy(
                rows[b],
                out_hbm.at[b_idx, pl.ds(pos_base + cc * _CHUNK, _CHUNK)],
                osems[b],
            )

        for step in range(n_chunks + 1):
            if step < n_chunks:
                b = step % _NBUF
                if step >= _NBUF:
                    out_copy(step - _NBUF, b).wait()
                gather(step, b).start()
            if step >= 1:
                c = step - 1
                b = c % _NBUF
                gather(c, b).wait()
                wrow_base = (c % chunks_per_seq) * _CHUNK

                @pl.loop(0, _CHUNK)
                def _row_loop(r, _b=b, _wb=wrow_base):
                    @plsc.parallel_loop(0, d, step=_LANES, unroll=4)
                    def _lane_loop(j):
                        plsc.addupdate(
                            rows[_b].at[r, pl.ds(j, _LANES)],
                            wpe_v[_wb + r, pl.ds(j, _LANES)],
                        )

                out_copy(c, b).start()

        for c in range(n_chunks - _NBUF, n_chunks):
            out_copy(c, c % _NBUF).wait()

    return emb_kernel(wte, labels_flat, wpe)


def kernel(hidden, labels, wte, wpe):
    b, s = labels.shape
    labels_flat = labels.reshape(b * s)
    emb = _emb_lookup(labels_flat, wte, wpe, b, s)
    attention_mask = labels != 0
    return (hidden, emb, labels, attention_mask)


# speedup vs baseline: 2.5488x; 1.1454x over previous
import functools

import jax
import jax.numpy as jnp
from jax import lax
from jax.experimental import pallas as pl
from jax.experimental.pallas import tpu as pltpu
from jax.experimental.pallas import tpu_sc as plsc

_NUM_CORES = 2
_NUM_SUBCORES = 16
_NUM_WORKERS = _NUM_CORES * _NUM_SUBCORES
_LANES = 16
_NBUF = 3
_CHUNK = 16


def _emb_lookup(labels_flat, wte, wpe, batch, seq_len):
    n = labels_flat.shape[0]
    _, d = wte.shape
    pos_per_w = seq_len // _NUM_WORKERS
    per_w = n // _NUM_WORKERS
    n_chunks = per_w // _CHUNK
    chunks_per_seq = pos_per_w // _CHUNK

    mesh = plsc.VectorSubcoreMesh(core_axis_name="c", subcore_axis_name="s")

    @functools.partial(
        pl.kernel,
        mesh=mesh,
        out_type=jax.ShapeDtypeStruct((batch, seq_len, d), jnp.float32),
        scratch_types=(
            [pltpu.VMEM((per_w,), jnp.int32),
             pltpu.VMEM((pos_per_w, d), jnp.float32)]
            + [pltpu.VMEM((_CHUNK, d), jnp.float32) for _ in range(_NBUF)]
            + [pltpu.SemaphoreType.DMA for _ in range(2 * _NBUF)]
        ),
    )
    def emb_kernel(wte_hbm, idx_hbm, wpe_hbm, out_hbm, idx_v, wpe_v, *bufs_and_sems):
        rows = bufs_and_sems[0:_NBUF]
        isems = bufs_and_sems[_NBUF:2 * _NBUF]
        osems = bufs_and_sems[2 * _NBUF:3 * _NBUF]

        wid = lax.axis_index("s") * _NUM_CORES + lax.axis_index("c")
        pos_base = wid * pos_per_w

        pltpu.sync_copy(wpe_hbm.at[pl.ds(pos_base, pos_per_w)], wpe_v)
        for b_idx in range(batch):
            pltpu.sync_copy(
                idx_hbm.at[pl.ds(b_idx * seq_len + pos_base, pos_per_w)],
                idx_v.at[pl.ds(b_idx * pos_per_w, pos_per_w)],
            )

        def gather(c, b):
            return pltpu.make_async_copy(
                wte_hbm.at[idx_v.at[pl.ds(c * _CHUNK, _CHUNK)]], rows[b], isems[b]
            )

        def out_copy(c, b):
            b_idx, cc = divmod(c, chunks_per_seq)
            return pltpu.make_async_copy(
                rows[b],
                out_hbm.at[b_idx, pl.ds(pos_base + cc * _CHUNK, _CHUNK)],
                osems[b],
            )

        for step in range(n_chunks + 1):
            if step < n_chunks:
                b = step % _NBUF
                if step >= _NBUF:
                    out_copy(step - _NBUF, b).wait()
                gather(step, b).start()
            if step >= 1:
                c = step - 1
                b = c % _NBUF
                gather(c, b).wait()
                wrow_base = (c % chunks_per_seq) * _CHUNK

                @pl.loop(0, _CHUNK)
                def _row_loop(r, _b=b, _wb=wrow_base):
                    @plsc.parallel_loop(0, d, step=_LANES, unroll=4)
                    def _lane_loop(j):
                        plsc.addupdate(
                            rows[_b].at[r, pl.ds(j, _LANES)],
                            wpe_v[_wb + r, pl.ds(j, _LANES)],
                        )

                out_copy(c, b).start()

        for c in range(n_chunks - _NBUF, n_chunks):
            out_copy(c, c % _NBUF).wait()

    return emb_kernel(wte, labels_flat, wpe)


def _tc_passthrough(x, n_blocks=8):
    bs, s, d = x.shape
    blk = (bs, s // n_blocks, d)

    def body(x_ref, o_ref):
        o_ref[...] = x_ref[...]

    return pl.pallas_call(
        body,
        out_shape=jax.ShapeDtypeStruct(x.shape, x.dtype),
        grid=(n_blocks,),
        in_specs=[pl.BlockSpec(blk, lambda i: (0, i, 0))],
        out_specs=pl.BlockSpec(blk, lambda i: (0, i, 0)),
    )(x)


def kernel(hidden, labels, wte, wpe):
    b, s = labels.shape
    labels_flat = labels.reshape(b * s)
    emb = _emb_lookup(labels_flat, wte, wpe, b, s)
    hidden_out = _tc_passthrough(hidden)
    attention_mask = labels != 0
    return (hidden_out, emb, labels, attention_mask)


# speedup vs baseline: 2.5840x; 1.0138x over previous
import functools

import jax
import jax.numpy as jnp
from jax import lax
from jax.experimental import pallas as pl
from jax.experimental.pallas import tpu as pltpu
from jax.experimental.pallas import tpu_sc as plsc

_NUM_CORES = 2
_NUM_SUBCORES = 16
_NUM_WORKERS = _NUM_CORES * _NUM_SUBCORES
_LANES = 16
_NBUF = 3
_CHUNK = 16


def _emb_lookup(labels_flat, wte, wpe, batch, seq_len):
    n = labels_flat.shape[0]
    _, d = wte.shape
    pos_per_w = seq_len // _NUM_WORKERS
    per_w = n // _NUM_WORKERS
    n_chunks = per_w // _CHUNK
    chunks_per_seq = pos_per_w // _CHUNK

    mesh = plsc.VectorSubcoreMesh(core_axis_name="c", subcore_axis_name="s")

    @functools.partial(
        pl.kernel,
        mesh=mesh,
        out_type=jax.ShapeDtypeStruct((batch, seq_len, d), jnp.float32),
        scratch_types=(
            [pltpu.VMEM((per_w,), jnp.int32),
             pltpu.VMEM((pos_per_w, d), jnp.float32)]
            + [pltpu.VMEM((_CHUNK, d), jnp.float32) for _ in range(_NBUF)]
            + [pltpu.SemaphoreType.DMA for _ in range(2 * _NBUF)]
        ),
    )
    def emb_kernel(wte_hbm, idx_hbm, wpe_hbm, out_hbm, idx_v, wpe_v, *bufs_and_sems):
        rows = bufs_and_sems[0:_NBUF]
        isems = bufs_and_sems[_NBUF:2 * _NBUF]
        osems = bufs_and_sems[2 * _NBUF:3 * _NBUF]

        wid = lax.axis_index("s") * _NUM_CORES + lax.axis_index("c")
        pos_base = wid * pos_per_w

        wpe_load = pltpu.make_async_copy(
            wpe_hbm.at[pl.ds(pos_base, pos_per_w)], wpe_v, osems[0]
        )
        wpe_load.start()
        idx_loads = [
            pltpu.make_async_copy(
                idx_hbm.at[pl.ds(b_idx * seq_len + pos_base, pos_per_w)],
                idx_v.at[pl.ds(b_idx * pos_per_w, pos_per_w)],
                isems[0],
            )
            for b_idx in range(batch)
        ]
        for cp in idx_loads:
            cp.start()
        for cp in idx_loads:
            cp.wait()

        def gather(c, b):
            return pltpu.make_async_copy(
                wte_hbm.at[idx_v.at[pl.ds(c * _CHUNK, _CHUNK)]], rows[b], isems[b]
            )

        def out_copy(c, b):
            b_idx, cc = divmod(c, chunks_per_seq)
            return pltpu.make_async_copy(
                rows[b],
                out_hbm.at[b_idx, pl.ds(pos_base + cc * _CHUNK, _CHUNK)],
                osems[b],
            )

        for step in range(n_chunks + 1):
            if step < n_chunks:
                b = step % _NBUF
                if step >= _NBUF:
                    out_copy(step - _NBUF, b).wait()
                gather(step, b).start()
            if step >= 1:
                c = step - 1
                b = c % _NBUF
                if c == 0:
                    wpe_load.wait()
                gather(c, b).wait()
                wrow_base = (c % chunks_per_seq) * _CHUNK

                @pl.loop(0, _CHUNK)
                def _row_loop(r, _b=b, _wb=wrow_base):
                    @plsc.parallel_loop(0, d, step=_LANES, unroll=8)
                    def _lane_loop(j):
                        plsc.addupdate(
                            rows[_b].at[r, pl.ds(j, _LANES)],
                            wpe_v[_wb + r, pl.ds(j, _LANES)],
                        )

                out_copy(c, b).start()

        for c in range(n_chunks - _NBUF, n_chunks):
            out_copy(c, c % _NBUF).wait()

    return emb_kernel(wte, labels_flat, wpe)


def _tc_passthrough(x, n_blocks=8):
    bs, s, d = x.shape
    blk = (bs, s // n_blocks, d)

    def body(x_ref, o_ref):
        o_ref[...] = x_ref[...]

    return pl.pallas_call(
        body,
        out_shape=jax.ShapeDtypeStruct(x.shape, x.dtype),
        grid=(n_blocks,),
        in_specs=[pl.BlockSpec(blk, lambda i: (0, i, 0))],
        out_specs=pl.BlockSpec(blk, lambda i: (0, i, 0)),
    )(x)


def kernel(hidden, labels, wte, wpe):
    b, s = labels.shape
    labels_flat = labels.reshape(b * s)
    emb = _emb_lookup(labels_flat, wte, wpe, b, s)
    hidden_out = _tc_passthrough(hidden)
    attention_mask = labels != 0
    return (hidden_out, emb, labels, attention_mask)


# speedup vs baseline: 2.5887x; 1.0018x over previous
import functools

import jax
import jax.numpy as jnp
from jax import lax
from jax.experimental import pallas as pl
from jax.experimental.pallas import tpu as pltpu
from jax.experimental.pallas import tpu_sc as plsc

_NUM_CORES = 2
_NUM_SUBCORES = 16
_NUM_WORKERS = _NUM_CORES * _NUM_SUBCORES
_LANES = 16
_NBUF = 3
_CHUNK = 16


def _emb_lookup(labels_flat, wte, wpe, batch, seq_len):
    n = labels_flat.shape[0]
    _, d = wte.shape
    pos_per_w = seq_len // _NUM_WORKERS
    per_w = n // _NUM_WORKERS
    n_chunks = per_w // _CHUNK
    chunks_per_seq = pos_per_w // _CHUNK

    mesh = plsc.VectorSubcoreMesh(core_axis_name="c", subcore_axis_name="s")

    @functools.partial(
        pl.kernel,
        mesh=mesh,
        out_type=jax.ShapeDtypeStruct((batch, seq_len, d), jnp.float32),
        scratch_types=(
            [pltpu.VMEM((per_w,), jnp.int32),
             pltpu.VMEM((pos_per_w, d), jnp.float32)]
            + [pltpu.VMEM((_CHUNK, d), jnp.float32) for _ in range(_NBUF)]
            + [pltpu.SemaphoreType.DMA for _ in range(2 * _NBUF)]
        ),
    )
    def emb_kernel(wte_hbm, idx_hbm, wpe_hbm, out_hbm, idx_v, wpe_v, *bufs_and_sems):
        rows = bufs_and_sems[0:_NBUF]
        isems = bufs_and_sems[_NBUF:2 * _NBUF]
        osems = bufs_and_sems[2 * _NBUF:3 * _NBUF]

        wid = lax.axis_index("s") * _NUM_CORES + lax.axis_index("c")
        pos_base = wid * pos_per_w

        wpe_load = pltpu.make_async_copy(
            wpe_hbm.at[pl.ds(pos_base, pos_per_w)], wpe_v, osems[0]
        )
        wpe_load.start()
        idx_loads = [
            pltpu.make_async_copy(
                idx_hbm.at[pl.ds(b_idx * seq_len + pos_base, pos_per_w)],
                idx_v.at[pl.ds(b_idx * pos_per_w, pos_per_w)],
                isems[0],
            )
            for b_idx in range(batch)
        ]
        for cp in idx_loads:
            cp.start()
        for cp in idx_loads:
            cp.wait()

        def gather(c, b):
            return pltpu.make_async_copy(
                wte_hbm.at[idx_v.at[pl.ds(c * _CHUNK, _CHUNK)]], rows[b], isems[b]
            )

        def out_copy(c, b):
            b_idx, cc = divmod(c, chunks_per_seq)
            return pltpu.make_async_copy(
                rows[b],
                out_hbm.at[b_idx, pl.ds(pos_base + cc * _CHUNK, _CHUNK)],
                osems[b],
            )

        gather(0, 0).start()
        gather(1, 1).start()
        for c in range(n_chunks):
            b = c % _NBUF
            if c == 0:
                wpe_load.wait()
            gather(c, b).wait()
            wrow_base = (c % chunks_per_seq) * _CHUNK

            @pl.loop(0, _CHUNK)
            def _row_loop(r, _b=b, _wb=wrow_base):
                @plsc.parallel_loop(0, d, step=_LANES, unroll=8)
                def _lane_loop(j):
                    plsc.addupdate(
                        rows[_b].at[r, pl.ds(j, _LANES)],
                        wpe_v[_wb + r, pl.ds(j, _LANES)],
                    )

            out_copy(c, b).start()
            nxt = c + 2
            if nxt < n_chunks:
                nb = nxt % _NBUF
                if nxt >= _NBUF:
                    out_copy(nxt - _NBUF, nb).wait()
                gather(nxt, nb).start()

        for c in range(n_chunks - _NBUF, n_chunks):
            out_copy(c, c % _NBUF).wait()

    return emb_kernel(wte, labels_flat, wpe)


def _tc_passthrough(x, n_blocks=8):
    bs, s, d = x.shape
    blk = (bs, s // n_blocks, d)

    def body(x_ref, o_ref):
        o_ref[...] = x_ref[...]

    return pl.pallas_call(
        body,
        out_shape=jax.ShapeDtypeStruct(x.shape, x.dtype),
        grid=(n_blocks,),
        in_specs=[pl.BlockSpec(blk, lambda i: (0, i, 0))],
        out_specs=pl.BlockSpec(blk, lambda i: (0, i, 0)),
    )(x)


def kernel(hidden, labels, wte, wpe):
    b, s = labels.shape
    labels_flat = labels.reshape(b * s)
    emb = _emb_lookup(labels_flat, wte, wpe, b, s)
    hidden_out = _tc_passthrough(hidden)
    attention_mask = labels != 0
    return (hidden_out, emb, labels, attention_mask)
